# baseline (speedup 1.0000x reference)
import jax
import jax.numpy as jnp
from jax import lax
from jax.experimental import pallas as pl
from jax.experimental.pallas import tpu as pltpu
from jax.experimental.pallas import tpu_sc as plsc

VOCAB = 1000000
DIM = 64
BATCH = 16384

_INFO = plsc.get_sparse_core_info()
_NC = _INFO.num_cores
_NS = _INFO.num_subcores
_NW = _NC * _NS
_L = _INFO.num_lanes

_ROWS_PER_W = BATCH // _NW
_CHUNK = 128
_NCHUNK = _ROWS_PER_W // _CHUNK
_W2 = 2 * DIM


def _sc_kernel(cw_hbm, xw_hbm, ctab_hbm, xtab_hbm, out_hbm,
               craw_v, xraw_v, chalf_v, xhalf_v, crows_v, xrows_v,
               tbuf_v, out_v, sems):
    wid = lax.axis_index("s") * _NC + lax.axis_index("c")
    base = wid * _ROWS_PER_W

    pltpu.sync_copy(cw_hbm.at[wid], craw_v)
    pltpu.sync_copy(xw_hbm.at[wid], xraw_v)

    for j in range(_NCHUNK):
        for t in range(_CHUNK // _L):
            s = pl.ds(t * _L, _L)
            chalf_v[j, s] = lax.shift_right_logical(craw_v[j, s], 1)
            xhalf_v[j, s] = lax.shift_right_logical(xraw_v[j, s], 1)

    pending = {}

    def fire(j):
        b = j % 2
        pending[j] = (
            pltpu.async_copy(ctab_hbm.at[chalf_v.at[j]], crows_v.at[b],
                             sems.at[b]),
            pltpu.async_copy(xtab_hbm.at[xhalf_v.at[j]], xrows_v.at[b],
                             sems.at[b]),
        )

    tidx = (_L + 1) * lax.iota(jnp.int32, _L)

    fire(0)
    for j in range(_NCHUNK):
        b = j % 2
        if j + 1 < _NCHUNK:
            fire(j + 1)
        ca, cb = pending.pop(j)
        ca.wait()
        cb.wait()

        def body(g, carry, j=j, b=b):
            r0 = g * _L
            pc16 = (craw_v[j, pl.ds(r0, _L)] & 1) * DIM
            px16 = (xraw_v[j, pl.ds(r0, _L)] & 1) * DIM
            for i in range(_L):
                r = r0 + i
                pc = pc16[i]
                px = px16[i]
                acc = None
                for q in range(DIM // _L):
                    cv = crows_v[b, r, pl.ds(pc + q * _L, _L)]
                    xv = xrows_v[b, r, pl.ds(px + q * _L, _L)]
                    t = cv * xv
                    acc = t if acc is None else acc + t
                tbuf_v[pl.ds(i * (_L + 1), _L)] = acc
            tot = jnp.zeros((_L,), jnp.float32)
            for q in range(_L):
                tot = tot + plsc.load_gather(tbuf_v, [tidx + q])
            out_v[pl.ds(j * _CHUNK + r0, _L)] = tot
            return carry

        lax.fori_loop(0, _CHUNK // _L, body, 0, unroll=False)

    pltpu.sync_copy(out_v, out_hbm.at[pl.ds(base, _ROWS_PER_W)])


@jax.jit
def kernel(center_words, context_words, center_table, context_table):
    cw = center_words.astype(jnp.int32).reshape(_NW, _NCHUNK, _CHUNK)
    xw = context_words.astype(jnp.int32).reshape(_NW, _NCHUNK, _CHUNK)
    ct = center_table.reshape(VOCAB // 2, _W2)
    xt = context_table.reshape(VOCAB // 2, _W2)
    mesh = plsc.VectorSubcoreMesh(core_axis_name="c", subcore_axis_name="s")
    run = pl.kernel(
        _sc_kernel,
        out_type=jax.ShapeDtypeStruct((BATCH,), jnp.float32),
        mesh=mesh,
        scratch_types=[
            pltpu.VMEM((_NCHUNK, _CHUNK), jnp.int32),
            pltpu.VMEM((_NCHUNK, _CHUNK), jnp.int32),
            pltpu.VMEM((_NCHUNK, _CHUNK), jnp.int32),
            pltpu.VMEM((_NCHUNK, _CHUNK), jnp.int32),
            pltpu.VMEM((2, _CHUNK, _W2), jnp.float32),
            pltpu.VMEM((2, _CHUNK, _W2), jnp.float32),
            pltpu.VMEM((_L * (_L + 1),), jnp.float32),
            pltpu.VMEM((_ROWS_PER_W,), jnp.float32),
            pltpu.SemaphoreType.DMA((2,)),
        ],
        compiler_params=pltpu.CompilerParams(needs_layout_passes=False),
    )
    return run(cw, xw, ct, xt)
